# baseline (device time: 1364518 ns/iter reference)
import jax
import jax.numpy as jnp
from jax import lax
from jax.experimental import pallas as pl
from jax.experimental.pallas import tpu as pltpu

N_DEV = 4


def _ring_all_reduce(x):
    t, d = x.shape
    assert t % N_DEV == 0
    chunk = t // N_DEV

    def body(x_ref, out_ref, comm_ref, send_sems, recv_sems):
        my = lax.axis_index("i")
        left = lax.rem(my - 1 + N_DEV, N_DEV)
        right = lax.rem(my + 1, N_DEV)

        barrier_sem = pltpu.get_barrier_semaphore()
        for nbr in (left, right):
            pl.semaphore_signal(
                barrier_sem, inc=1,
                device_id=(nbr,), device_id_type=pl.DeviceIdType.MESH,
            )
        pl.semaphore_wait(barrier_sem, 2)

        out_ref[:, :] = x_ref[:, :]

        for h in range(N_DEV - 1):
            send_c = lax.rem(my - h + N_DEV, N_DEV)
            recv_c = lax.rem(my - h - 1 + N_DEV, N_DEV)
            rdma = pltpu.make_async_remote_copy(
                src_ref=out_ref.at[pl.ds(send_c * chunk, chunk), :],
                dst_ref=comm_ref.at[h],
                send_sem=send_sems.at[h],
                recv_sem=recv_sems.at[h],
                device_id=(right,),
                device_id_type=pl.DeviceIdType.MESH,
            )
            rdma.start()
            rdma.wait()
            out_ref[pl.ds(recv_c * chunk, chunk), :] += comm_ref[h]

        for h in range(N_DEV - 1):
            send_c = lax.rem(my + 1 - h + N_DEV, N_DEV)
            rdma = pltpu.make_async_remote_copy(
                src_ref=out_ref.at[pl.ds(send_c * chunk, chunk), :],
                dst_ref=out_ref.at[pl.ds(send_c * chunk, chunk), :],
                send_sem=send_sems.at[N_DEV - 1 + h],
                recv_sem=recv_sems.at[N_DEV - 1 + h],
                device_id=(right,),
                device_id_type=pl.DeviceIdType.MESH,
            )
            rdma.start()
            rdma.wait()

    return pl.pallas_call(
        body,
        out_shape=jax.ShapeDtypeStruct((t, d), x.dtype),
        in_specs=[pl.BlockSpec(memory_space=pltpu.VMEM)],
        out_specs=pl.BlockSpec(memory_space=pltpu.VMEM),
        scratch_shapes=[
            pltpu.VMEM((N_DEV - 1, chunk, d), x.dtype),
            pltpu.SemaphoreType.DMA((2 * (N_DEV - 1),)),
            pltpu.SemaphoreType.DMA((2 * (N_DEV - 1),)),
        ],
        compiler_params=pltpu.CompilerParams(collective_id=0),
    )(x)


def kernel(ids, E):
    my = lax.axis_index("i")
    v_per = E.shape[0]
    local = ids - my * v_per
    mask = (local >= 0) & (local < v_per)
    safe = jnp.where(mask, local, 0)
    partial = jnp.where(mask[:, None], jnp.take(E, safe, axis=0), 0.0)
    return _ring_all_reduce(partial)


# device time: 214915 ns/iter; 6.3491x vs baseline; 6.3491x over previous
import jax
import jax.numpy as jnp
from jax import lax
from jax.experimental import pallas as pl
from jax.experimental.pallas import tpu as pltpu

N_DEV = 4


def kernel(ids, E):
    t_tot = ids.shape[0]
    v_per, d = E.shape
    assert t_tot % N_DEV == 0
    chunk = t_tot // N_DEV

    my_out = lax.axis_index("i")
    maskf = (
        (ids >= my_out * v_per) & (ids < (my_out + 1) * v_per)
    ).astype(jnp.float32)[:, None]

    def body(ids_ref, e_ref, mask_ref, out_ref, comm_ref, send_sems,
             recv_sems, gather_sem):
        my = lax.axis_index("i")
        left = lax.rem(my - 1 + N_DEV, N_DEV)
        right = lax.rem(my + 1, N_DEV)

        barrier_sem = pltpu.get_barrier_semaphore()
        for nbr in (left, right):
            pl.semaphore_signal(
                barrier_sem, inc=1,
                device_id=(nbr,), device_id_type=pl.DeviceIdType.MESH,
            )
        pl.semaphore_wait(barrier_sem, 2)

        base = my * v_per

        def issue(t, _):
            lid = jnp.clip(ids_ref[t] - base, 0, v_per - 1)
            pltpu.make_async_copy(
                e_ref.at[pl.ds(lid, 1), :],
                out_ref.at[pl.ds(t, 1), :],
                gather_sem,
            ).start()
            return 0

        lax.fori_loop(0, t_tot, issue, 0)

        def drain(t, _):
            pltpu.make_async_copy(
                e_ref.at[pl.ds(0, 1), :],
                out_ref.at[pl.ds(0, 1), :],
                gather_sem,
            ).wait()
            return 0

        lax.fori_loop(0, t_tot, drain, 0)
        out_ref[:, :] = out_ref[:, :] * mask_ref[:, :]

        for h in range(N_DEV - 1):
            send_c = lax.rem(my - h + N_DEV, N_DEV)
            recv_c = lax.rem(my - h - 1 + N_DEV, N_DEV)
            rdma = pltpu.make_async_remote_copy(
                src_ref=out_ref.at[pl.ds(send_c * chunk, chunk), :],
                dst_ref=comm_ref.at[h],
                send_sem=send_sems.at[h],
                recv_sem=recv_sems.at[h],
                device_id=(right,),
                device_id_type=pl.DeviceIdType.MESH,
            )
            rdma.start()
            rdma.wait()
            out_ref[pl.ds(recv_c * chunk, chunk), :] += comm_ref[h]

        for h in range(N_DEV - 1):
            send_c = lax.rem(my + 1 - h + N_DEV, N_DEV)
            rdma = pltpu.make_async_remote_copy(
                src_ref=out_ref.at[pl.ds(send_c * chunk, chunk), :],
                dst_ref=out_ref.at[pl.ds(send_c * chunk, chunk), :],
                send_sem=send_sems.at[N_DEV - 1 + h],
                recv_sem=recv_sems.at[N_DEV - 1 + h],
                device_id=(right,),
                device_id_type=pl.DeviceIdType.MESH,
            )
            rdma.start()
            rdma.wait()

    return pl.pallas_call(
        body,
        out_shape=jax.ShapeDtypeStruct((t_tot, d), E.dtype),
        in_specs=[
            pl.BlockSpec(memory_space=pltpu.SMEM),
            pl.BlockSpec(memory_space=pl.ANY),
            pl.BlockSpec(memory_space=pltpu.VMEM),
        ],
        out_specs=pl.BlockSpec(memory_space=pltpu.VMEM),
        scratch_shapes=[
            pltpu.VMEM((N_DEV - 1, chunk, d), E.dtype),
            pltpu.SemaphoreType.DMA((2 * (N_DEV - 1),)),
            pltpu.SemaphoreType.DMA((2 * (N_DEV - 1),)),
            pltpu.SemaphoreType.DMA,
        ],
        compiler_params=pltpu.CompilerParams(collective_id=0),
    )(ids, E, maskf)


# device time: 135474 ns/iter; 10.0722x vs baseline; 1.5864x over previous
import jax
import jax.numpy as jnp
from jax import lax
from jax.experimental import pallas as pl
from jax.experimental.pallas import tpu as pltpu

N_DEV = 4
DELTAS = (0, 1, 3, 2)


def kernel(ids, E):
    t_tot = ids.shape[0]
    v_per, d = E.shape
    chunk = t_tot // N_DEV
    half = d // 2

    my_out = lax.axis_index("i")
    maskf = (
        (ids >= my_out * v_per) & (ids < (my_out + 1) * v_per)
    ).astype(jnp.float32)[:, None]

    def body(ids_ref, e_ref, mask_ref, out_ref, comm_r, comm_l,
             send_r, recv_r, send_l, recv_l, gather_sems):
        my = lax.axis_index("i")
        left = lax.rem(my - 1 + N_DEV, N_DEV)
        right = lax.rem(my + 1, N_DEV)

        barrier_sem = pltpu.get_barrier_semaphore()
        for nbr in (left, right):
            pl.semaphore_signal(
                barrier_sem, inc=1,
                device_id=(nbr,), device_id_type=pl.DeviceIdType.MESH,
            )
        pl.semaphore_wait(barrier_sem, 2)

        base = my * v_per

        def issue(p):
            c = lax.rem(my + DELTAS[p], N_DEV)

            def step(j, _):
                t = c * chunk + j
                lid = jnp.clip(ids_ref[t] - base, 0, v_per - 1)
                pltpu.make_async_copy(
                    e_ref.at[pl.ds(lid, 1), :],
                    out_ref.at[pl.ds(t, 1), :],
                    gather_sems.at[p],
                ).start()
                return 0

            lax.fori_loop(0, chunk, step, 0, unroll=8)

        def drain(p):
            c = lax.rem(my + DELTAS[p], N_DEV)

            def step(j, _):
                pltpu.make_async_copy(
                    e_ref.at[pl.ds(0, 1), :],
                    out_ref.at[pl.ds(0, 1), :],
                    gather_sems.at[p],
                ).wait()
                return 0

            lax.fori_loop(0, chunk, step, 0, unroll=8)
            r = pl.ds(c * chunk, chunk)
            out_ref[r, :] = out_ref[r, :] * mask_ref[r, :]

        def hop_rdma(h, c_r, c_l, sem_idx, rs):
            rr = pl.ds(c_r * chunk, chunk)
            rl = pl.ds(c_l * chunk, chunk)
            rdma_r = pltpu.make_async_remote_copy(
                src_ref=out_ref.at[rr, pl.ds(0, half)],
                dst_ref=(comm_r.at[h] if rs
                         else out_ref.at[rr, pl.ds(0, half)]),
                send_sem=send_r.at[sem_idx],
                recv_sem=recv_r.at[sem_idx],
                device_id=(right,),
                device_id_type=pl.DeviceIdType.MESH,
            )
            rdma_l = pltpu.make_async_remote_copy(
                src_ref=out_ref.at[rl, pl.ds(half, half)],
                dst_ref=(comm_l.at[h] if rs
                         else out_ref.at[rl, pl.ds(half, half)]),
                send_sem=send_l.at[sem_idx],
                recv_sem=recv_l.at[sem_idx],
                device_id=(left,),
                device_id_type=pl.DeviceIdType.MESH,
            )
            rdma_r.start()
            rdma_l.start()
            return rdma_r, rdma_l

        issue(0)
        drain(0)
        rdmas = hop_rdma(0, my, my, 0, rs=True)
        issue(1)
        issue(2)
        issue(3)
        drain(1)
        drain(2)
        for h in range(N_DEV - 1):
            acc_r = lax.rem(my - h - 1 + N_DEV, N_DEV)
            acc_l = lax.rem(my + h + 1, N_DEV)
            rdmas[0].wait()
            out_ref[pl.ds(acc_r * chunk, chunk), pl.ds(0, half)] += comm_r[h]
            rdmas[1].wait()
            out_ref[pl.ds(acc_l * chunk, chunk), pl.ds(half, half)] += comm_l[h]
            if h == 0:
                drain(3)
            if h < N_DEV - 2:
                rdmas = hop_rdma(h + 1, acc_r, acc_l, h + 1, rs=True)

        for h in range(N_DEV - 1):
            c_r = lax.rem(my + 1 - h + N_DEV, N_DEV)
            c_l = lax.rem(my - 1 + h + N_DEV, N_DEV)
            rdmas = hop_rdma(h, c_r, c_l, N_DEV - 1 + h, rs=False)
            rdmas[0].wait()
            rdmas[1].wait()

    return pl.pallas_call(
        body,
        out_shape=jax.ShapeDtypeStruct((t_tot, d), E.dtype),
        in_specs=[
            pl.BlockSpec(memory_space=pltpu.SMEM),
            pl.BlockSpec(memory_space=pl.ANY),
            pl.BlockSpec(memory_space=pltpu.VMEM),
        ],
        out_specs=pl.BlockSpec(memory_space=pltpu.VMEM),
        scratch_shapes=[
            pltpu.VMEM((N_DEV - 1, chunk, half), E.dtype),
            pltpu.VMEM((N_DEV - 1, chunk, half), E.dtype),
            pltpu.SemaphoreType.DMA((2 * (N_DEV - 1),)),
            pltpu.SemaphoreType.DMA((2 * (N_DEV - 1),)),
            pltpu.SemaphoreType.DMA((2 * (N_DEV - 1),)),
            pltpu.SemaphoreType.DMA((2 * (N_DEV - 1),)),
            pltpu.SemaphoreType.DMA((N_DEV,)),
        ],
        compiler_params=pltpu.CompilerParams(collective_id=0),
    )(ids, E, maskf)
